# baseline (speedup 1.0000x reference)
import functools

import jax
import jax.numpy as jnp
import numpy as np
from jax.experimental import pallas as pl

IN_FEATURES = 2048
OUT_FEATURES = 2048
NUM_EXPERTS = 8
LORA_RANK = 16
SPARSEGEN_LAMBDA = 0.5
LORA_SCALING = 2.0

LANES = 128
BT = 256


def _body(x_ref, wt_ref, b_ref, rw_ref, rb_ref, acat_ref, bcat_ref, rep_ref,
          o_ref):
    x = x_ref[...]
    xb = x.astype(jnp.bfloat16)

    z = jnp.dot(x, rw_ref[...], preferred_element_type=jnp.float32)
    z = (z + rb_ref[...]) * (1.0 / (1.0 - SPARSEGEN_LAMBDA))
    lane = jax.lax.broadcasted_iota(jnp.int32, z.shape, dimension=1)
    mask = lane < NUM_EXPERTS
    active = mask.astype(jnp.float32)
    tau = jnp.zeros((z.shape[0], 1), jnp.float32)
    for _ in range(NUM_EXPERTS):
        n = jnp.sum(active, axis=-1, keepdims=True)
        s = jnp.sum(z * active, axis=-1, keepdims=True)
        tau = (s - 1.0) / n
        active = jnp.where(mask & (z > tau), 1.0, 0.0)
    w = jnp.where(mask, jnp.maximum(z - tau, 0.0), 0.0)

    wrep = jnp.dot(w, rep_ref[...], preferred_element_type=jnp.float32)
    h = jnp.dot(xb, acat_ref[...], preferred_element_type=jnp.float32)
    eo = jnp.dot((wrep * h).astype(jnp.bfloat16), bcat_ref[...],
                 preferred_element_type=jnp.float32)

    base = jnp.dot(xb, wt_ref[...], preferred_element_type=jnp.float32)
    o_ref[...] = base + b_ref[...] + LORA_SCALING * eo


@functools.partial(jax.jit, static_argnames=())
def kernel(x, W_base, b_base, router_W, router_b, lora_A, lora_B):
    T = x.shape[0]
    Wt = W_base.T.astype(jnp.bfloat16)
    b2 = b_base.reshape(1, OUT_FEATURES)
    rw = jnp.zeros((IN_FEATURES, LANES), jnp.float32).at[:, :NUM_EXPERTS].set(
        router_W)
    rb = jnp.zeros((1, LANES), jnp.float32).at[0, :NUM_EXPERTS].set(router_b)
    acat = lora_A.transpose(1, 0, 2).reshape(
        IN_FEATURES, NUM_EXPERTS * LORA_RANK).astype(jnp.bfloat16)
    bcat = lora_B.reshape(NUM_EXPERTS * LORA_RANK,
                          OUT_FEATURES).astype(jnp.bfloat16)
    rep = np.zeros((LANES, LANES), np.float32)
    for e in range(NUM_EXPERTS):
        rep[e, e * LORA_RANK:(e + 1) * LORA_RANK] = 1.0
    rep = jnp.asarray(rep)

    grid = (T // BT,)
    out = pl.pallas_call(
        _body,
        grid=grid,
        in_specs=[
            pl.BlockSpec((BT, IN_FEATURES), lambda i: (i, 0)),
            pl.BlockSpec((IN_FEATURES, OUT_FEATURES), lambda i: (0, 0)),
            pl.BlockSpec((1, OUT_FEATURES), lambda i: (0, 0)),
            pl.BlockSpec((IN_FEATURES, LANES), lambda i: (0, 0)),
            pl.BlockSpec((1, LANES), lambda i: (0, 0)),
            pl.BlockSpec((IN_FEATURES, NUM_EXPERTS * LORA_RANK),
                         lambda i: (0, 0)),
            pl.BlockSpec((NUM_EXPERTS * LORA_RANK, OUT_FEATURES),
                         lambda i: (0, 0)),
            pl.BlockSpec((LANES, LANES), lambda i: (0, 0)),
        ],
        out_specs=pl.BlockSpec((BT, OUT_FEATURES), lambda i: (i, 0)),
        out_shape=jax.ShapeDtypeStruct((T, OUT_FEATURES), jnp.float32),
    )(x, Wt, b2, rw, rb, acat, bcat, rep)
    return out


# speedup vs baseline: 1.1878x; 1.1878x over previous
import functools

import jax
import jax.numpy as jnp
import numpy as np
from jax.experimental import pallas as pl

IN_FEATURES = 2048
OUT_FEATURES = 2048
NUM_EXPERTS = 8
LORA_RANK = 16
SPARSEGEN_LAMBDA = 0.5
LORA_SCALING = 2.0

LANES = 128
BT = 1024


def _body(x_ref, wt_ref, b_ref, rw_ref, rb_ref, acat_ref, bcat_ref, rep_ref,
          o_ref):
    x = x_ref[...]
    xb = x.astype(jnp.bfloat16)

    z = jnp.dot(x, rw_ref[...], preferred_element_type=jnp.float32)
    zt = z[:, :NUM_EXPERTS].T
    zt = (zt + rb_ref[...][:, :1]) * (1.0 / (1.0 - SPARSEGEN_LAMBDA))
    active = jnp.ones(zt.shape, jnp.float32)
    tau = jnp.zeros((1, zt.shape[1]), jnp.float32)
    for _ in range(NUM_EXPERTS):
        n = jnp.sum(active, axis=0, keepdims=True)
        s = jnp.sum(zt * active, axis=0, keepdims=True)
        tau = (s - 1.0) / n
        active = jnp.where(zt > tau, 1.0, 0.0)
    w = jnp.maximum(zt - tau, 0.0).T

    wrep = jnp.dot(w, rep_ref[...], preferred_element_type=jnp.float32)
    h = jnp.dot(xb, acat_ref[...], preferred_element_type=jnp.float32)
    eo = jnp.dot((wrep * h).astype(jnp.bfloat16), bcat_ref[...],
                 preferred_element_type=jnp.float32)

    base = jnp.dot(xb, wt_ref[...], preferred_element_type=jnp.float32)
    o_ref[...] = base + b_ref[...] + LORA_SCALING * eo


@functools.partial(jax.jit, static_argnames=())
def kernel(x, W_base, b_base, router_W, router_b, lora_A, lora_B):
    T = x.shape[0]
    Wt = W_base.T.astype(jnp.bfloat16)
    b2 = b_base.reshape(1, OUT_FEATURES)
    rw = jnp.zeros((IN_FEATURES, LANES), jnp.float32).at[:, :NUM_EXPERTS].set(
        router_W)
    rb = jnp.broadcast_to(router_b.reshape(NUM_EXPERTS, 1),
                          (NUM_EXPERTS, LANES))
    acat = lora_A.transpose(1, 0, 2).reshape(
        IN_FEATURES, NUM_EXPERTS * LORA_RANK).astype(jnp.bfloat16)
    bcat = lora_B.reshape(NUM_EXPERTS * LORA_RANK,
                          OUT_FEATURES).astype(jnp.bfloat16)
    rep = np.zeros((NUM_EXPERTS, LANES), np.float32)
    for e in range(NUM_EXPERTS):
        rep[e, e * LORA_RANK:(e + 1) * LORA_RANK] = 1.0
    rep = jnp.asarray(rep)

    grid = (T // BT,)
    out = pl.pallas_call(
        _body,
        grid=grid,
        in_specs=[
            pl.BlockSpec((BT, IN_FEATURES), lambda i: (i, 0)),
            pl.BlockSpec((IN_FEATURES, OUT_FEATURES), lambda i: (0, 0)),
            pl.BlockSpec((1, OUT_FEATURES), lambda i: (0, 0)),
            pl.BlockSpec((IN_FEATURES, LANES), lambda i: (0, 0)),
            pl.BlockSpec((NUM_EXPERTS, LANES), lambda i: (0, 0)),
            pl.BlockSpec((IN_FEATURES, NUM_EXPERTS * LORA_RANK),
                         lambda i: (0, 0)),
            pl.BlockSpec((NUM_EXPERTS * LORA_RANK, OUT_FEATURES),
                         lambda i: (0, 0)),
            pl.BlockSpec((NUM_EXPERTS, LANES), lambda i: (0, 0)),
        ],
        out_specs=pl.BlockSpec((BT, OUT_FEATURES), lambda i: (i, 0)),
        out_shape=jax.ShapeDtypeStruct((T, OUT_FEATURES), jnp.float32),
    )(x, Wt, b2, rw, rb, acat, bcat, rep)
    return out


# speedup vs baseline: 1.1903x; 1.0021x over previous
import functools

import jax
import jax.numpy as jnp
import numpy as np
from jax.experimental import pallas as pl

IN_FEATURES = 2048
OUT_FEATURES = 2048
NUM_EXPERTS = 8
LORA_RANK = 16
SPARSEGEN_LAMBDA = 0.5
LORA_SCALING = 2.0

LANES = 128
BT = 1024


def _body(x_ref, wt_ref, b_ref, rw_ref, rb_ref, acat_ref, bcat_ref, rep_ref,
          o_ref):
    x = x_ref[...]
    xb = x.astype(jnp.bfloat16)

    z = jnp.dot(xb, rw_ref[...], preferred_element_type=jnp.float32)
    zt = z[:, :NUM_EXPERTS].T
    zt = (zt + rb_ref[...][:, :1]) * (1.0 / (1.0 - SPARSEGEN_LAMBDA))
    active = jnp.ones(zt.shape, jnp.float32)
    tau = jnp.zeros((1, zt.shape[1]), jnp.float32)
    for _ in range(NUM_EXPERTS):
        n = jnp.sum(active, axis=0, keepdims=True)
        s = jnp.sum(zt * active, axis=0, keepdims=True)
        tau = (s - 1.0) / n
        active = jnp.where(zt > tau, 1.0, 0.0)
    w = jnp.maximum(zt - tau, 0.0).T

    wrep = jnp.dot(w, rep_ref[...], preferred_element_type=jnp.float32)
    h = jnp.dot(xb, acat_ref[...], preferred_element_type=jnp.float32)
    eo = jnp.dot((wrep * h).astype(jnp.bfloat16), bcat_ref[...],
                 preferred_element_type=jnp.float32)

    base = jnp.dot(xb, wt_ref[...], preferred_element_type=jnp.float32)
    o_ref[...] = base + b_ref[...] + LORA_SCALING * eo


@functools.partial(jax.jit, static_argnames=())
def kernel(x, W_base, b_base, router_W, router_b, lora_A, lora_B):
    T = x.shape[0]
    Wt = W_base.T.astype(jnp.bfloat16)
    b2 = b_base.reshape(1, OUT_FEATURES)
    rw = jnp.zeros((IN_FEATURES, LANES), jnp.float32).at[:, :NUM_EXPERTS].set(
        router_W).astype(jnp.bfloat16)
    rb = jnp.broadcast_to(router_b.reshape(NUM_EXPERTS, 1),
                          (NUM_EXPERTS, LANES))
    acat = lora_A.transpose(1, 0, 2).reshape(
        IN_FEATURES, NUM_EXPERTS * LORA_RANK).astype(jnp.bfloat16)
    bcat = lora_B.reshape(NUM_EXPERTS * LORA_RANK,
                          OUT_FEATURES).astype(jnp.bfloat16)
    rep = np.zeros((NUM_EXPERTS, LANES), np.float32)
    for e in range(NUM_EXPERTS):
        rep[e, e * LORA_RANK:(e + 1) * LORA_RANK] = 1.0
    rep = jnp.asarray(rep)

    grid = (T // BT,)
    out = pl.pallas_call(
        _body,
        grid=grid,
        in_specs=[
            pl.BlockSpec((BT, IN_FEATURES), lambda i: (i, 0)),
            pl.BlockSpec((IN_FEATURES, OUT_FEATURES), lambda i: (0, 0)),
            pl.BlockSpec((1, OUT_FEATURES), lambda i: (0, 0)),
            pl.BlockSpec((IN_FEATURES, LANES), lambda i: (0, 0)),
            pl.BlockSpec((NUM_EXPERTS, LANES), lambda i: (0, 0)),
            pl.BlockSpec((IN_FEATURES, NUM_EXPERTS * LORA_RANK),
                         lambda i: (0, 0)),
            pl.BlockSpec((NUM_EXPERTS * LORA_RANK, OUT_FEATURES),
                         lambda i: (0, 0)),
            pl.BlockSpec((NUM_EXPERTS, LANES), lambda i: (0, 0)),
        ],
        out_specs=pl.BlockSpec((BT, OUT_FEATURES), lambda i: (i, 0)),
        out_shape=jax.ShapeDtypeStruct((T, OUT_FEATURES), jnp.float32),
    )(x, Wt, b2, rw, rb, acat, bcat, rep)
    return out


# speedup vs baseline: 1.1953x; 1.0042x over previous
import functools

import jax
import jax.numpy as jnp
import numpy as np
from jax.experimental import pallas as pl

IN_FEATURES = 2048
OUT_FEATURES = 2048
NUM_EXPERTS = 8
LORA_RANK = 16
SPARSEGEN_LAMBDA = 0.5
LORA_SCALING = 2.0

LANES = 128
NCAT = OUT_FEATURES + 2 * LANES
BT = 1024


def _body(x_ref, wcat_ref, b_ref, rb_ref, bcat_ref, rep_ref, o_ref):
    xb = x_ref[...].astype(jnp.bfloat16)

    big = jnp.dot(xb, wcat_ref[...], preferred_element_type=jnp.float32)
    base = big[:, :OUT_FEATURES]
    h = big[:, OUT_FEATURES:OUT_FEATURES + LANES]
    z = big[:, OUT_FEATURES + LANES:]

    zt = z[:, :NUM_EXPERTS].T
    zt = (zt + rb_ref[...][:, :1]) * (1.0 / (1.0 - SPARSEGEN_LAMBDA))
    active = jnp.ones(zt.shape, jnp.float32)
    tau = jnp.zeros((1, zt.shape[1]), jnp.float32)
    for _ in range(NUM_EXPERTS):
        n = jnp.sum(active, axis=0, keepdims=True)
        s = jnp.sum(zt * active, axis=0, keepdims=True)
        tau = (s - 1.0) / n
        active = jnp.where(zt > tau, 1.0, 0.0)
    w = jnp.maximum(zt - tau, 0.0).T

    wrep = jnp.dot(w, rep_ref[...], preferred_element_type=jnp.float32)
    eo = jnp.dot((wrep * h).astype(jnp.bfloat16), bcat_ref[...],
                 preferred_element_type=jnp.float32)

    o_ref[...] = base + b_ref[...] + LORA_SCALING * eo


@functools.partial(jax.jit, static_argnames=())
def kernel(x, W_base, b_base, router_W, router_b, lora_A, lora_B):
    T = x.shape[0]
    b2 = b_base.reshape(1, OUT_FEATURES)
    rw = jnp.zeros((IN_FEATURES, LANES), jnp.float32).at[:, :NUM_EXPERTS].set(
        router_W)
    rb = jnp.broadcast_to(router_b.reshape(NUM_EXPERTS, 1),
                          (NUM_EXPERTS, LANES))
    acat = lora_A.transpose(1, 0, 2).reshape(IN_FEATURES,
                                             NUM_EXPERTS * LORA_RANK)
    wcat = jnp.concatenate([W_base.T, acat, rw], axis=1).astype(jnp.bfloat16)
    bcat = lora_B.reshape(NUM_EXPERTS * LORA_RANK,
                          OUT_FEATURES).astype(jnp.bfloat16)
    rep = np.zeros((NUM_EXPERTS, LANES), np.float32)
    for e in range(NUM_EXPERTS):
        rep[e, e * LORA_RANK:(e + 1) * LORA_RANK] = 1.0
    rep = jnp.asarray(rep)

    grid = (T // BT,)
    out = pl.pallas_call(
        _body,
        grid=grid,
        in_specs=[
            pl.BlockSpec((BT, IN_FEATURES), lambda i: (i, 0)),
            pl.BlockSpec((IN_FEATURES, NCAT), lambda i: (0, 0)),
            pl.BlockSpec((1, OUT_FEATURES), lambda i: (0, 0)),
            pl.BlockSpec((NUM_EXPERTS, LANES), lambda i: (0, 0)),
            pl.BlockSpec((NUM_EXPERTS * LORA_RANK, OUT_FEATURES),
                         lambda i: (0, 0)),
            pl.BlockSpec((NUM_EXPERTS, LANES), lambda i: (0, 0)),
        ],
        out_specs=pl.BlockSpec((BT, OUT_FEATURES), lambda i: (i, 0)),
        out_shape=jax.ShapeDtypeStruct((T, OUT_FEATURES), jnp.float32),
    )(x, wcat, b2, rb, bcat, rep)
    return out


# speedup vs baseline: 1.2408x; 1.0381x over previous
import functools

import jax
import jax.numpy as jnp
import numpy as np
from jax.experimental import pallas as pl

IN_FEATURES = 2048
OUT_FEATURES = 2048
NUM_EXPERTS = 8
LORA_RANK = 16
SPARSEGEN_LAMBDA = 0.5
LORA_SCALING = 2.0

LANES = 128
NCAT = OUT_FEATURES + 2 * LANES
BT = 1024


def _body(x_ref, wcat_ref, b_ref, rb_ref, bcat_ref, rep_ref, o_ref):
    xb = x_ref[...].astype(jnp.bfloat16)

    big = jax.lax.dot_general(xb, wcat_ref[...], (((1,), (1,)), ((), ())),
                              preferred_element_type=jnp.float32)
    base = big[:, :OUT_FEATURES]
    h = big[:, OUT_FEATURES:OUT_FEATURES + LANES]
    z = big[:, OUT_FEATURES + LANES:]

    zt = z[:, :NUM_EXPERTS].T
    zt = (zt + rb_ref[...][:, :1]) * (1.0 / (1.0 - SPARSEGEN_LAMBDA))
    active = jnp.ones(zt.shape, jnp.float32)
    tau = jnp.zeros((1, zt.shape[1]), jnp.float32)
    for _ in range(NUM_EXPERTS):
        n = jnp.sum(active, axis=0, keepdims=True)
        s = jnp.sum(zt * active, axis=0, keepdims=True)
        tau = (s - 1.0) / n
        active = jnp.where(zt > tau, 1.0, 0.0)
    w = jnp.maximum(zt - tau, 0.0).T

    wrep = jnp.dot(w, rep_ref[...], preferred_element_type=jnp.float32)
    eo = jnp.dot((wrep * h).astype(jnp.bfloat16), bcat_ref[...],
                 preferred_element_type=jnp.float32)

    o_ref[...] = base + b_ref[...] + LORA_SCALING * eo


@functools.partial(jax.jit, static_argnames=())
def kernel(x, W_base, b_base, router_W, router_b, lora_A, lora_B):
    T = x.shape[0]
    b2 = b_base.reshape(1, OUT_FEATURES)
    rwt = jnp.zeros((LANES, IN_FEATURES), jnp.float32).at[:NUM_EXPERTS].set(
        router_W.T)
    rb = jnp.broadcast_to(router_b.reshape(NUM_EXPERTS, 1),
                          (NUM_EXPERTS, LANES))
    acat_t = lora_A.transpose(0, 2, 1).reshape(NUM_EXPERTS * LORA_RANK,
                                               IN_FEATURES)
    wcat = jnp.concatenate([W_base, acat_t, rwt], axis=0).astype(jnp.bfloat16)
    bcat = lora_B.reshape(NUM_EXPERTS * LORA_RANK,
                          OUT_FEATURES).astype(jnp.bfloat16)
    rep = np.zeros((NUM_EXPERTS, LANES), np.float32)
    for e in range(NUM_EXPERTS):
        rep[e, e * LORA_RANK:(e + 1) * LORA_RANK] = 1.0
    rep = jnp.asarray(rep)

    grid = (T // BT,)
    out = pl.pallas_call(
        _body,
        grid=grid,
        in_specs=[
            pl.BlockSpec((BT, IN_FEATURES), lambda i: (i, 0)),
            pl.BlockSpec((NCAT, IN_FEATURES), lambda i: (0, 0)),
            pl.BlockSpec((1, OUT_FEATURES), lambda i: (0, 0)),
            pl.BlockSpec((NUM_EXPERTS, LANES), lambda i: (0, 0)),
            pl.BlockSpec((NUM_EXPERTS * LORA_RANK, OUT_FEATURES),
                         lambda i: (0, 0)),
            pl.BlockSpec((NUM_EXPERTS, LANES), lambda i: (0, 0)),
        ],
        out_specs=pl.BlockSpec((BT, OUT_FEATURES), lambda i: (i, 0)),
        out_shape=jax.ShapeDtypeStruct((T, OUT_FEATURES), jnp.float32),
    )(x, wcat, b2, rb, bcat, rep)
    return out
